# baseline (speedup 1.0000x reference)
import functools

import jax
import jax.numpy as jnp
from jax import lax
from jax.experimental import pallas as pl
from jax.experimental.pallas import tpu as pltpu
from jax.experimental.pallas import tpu_sc as plsc

_INFO = plsc.get_sparse_core_info()
_NC, _NS, _L = _INFO.num_cores, _INFO.num_subcores, _INFO.num_lanes
_NW = _NC * _NS

_BT = 128
_FT = 8


def _make_kernel(B, NNZ, V, D, idx_scale):
    assert B == _BT * _NW
    n_blocks = NNZ // _FT
    n_tail = NNZ - n_blocks * _FT
    c_tiles = D // _FT

    @functools.partial(
        pl.kernel,
        out_type=jax.ShapeDtypeStruct((NNZ, c_tiles, _NW, _FT, _BT),
                                      jnp.float32),
        mesh=plsc.VectorSubcoreMesh(core_axis_name="c", subcore_axis_name="s"),
        scratch_types=[
            pltpu.VMEM((_FT, _BT), jnp.int32),
            pltpu.VMEM((_FT, _BT), jnp.float32),
            pltpu.VMEM((_FT, _BT, D), jnp.float32),
            pltpu.VMEM((_FT, c_tiles, _FT, _BT), jnp.float32),
            pltpu.SemaphoreType.DMA,
            pltpu.SemaphoreType.DMA,
        ],
        compiler_params=pltpu.CompilerParams(
            use_tc_tiling_on_sc=False, needs_layout_passes=False
        ),
    )
    def k(table_hbm, xq_hbm, vq_hbm, out_hbm, idx_v, val_v, rows_v, out_v,
          gsem, osem):
        w = lax.axis_index("s") * _NC + lax.axis_index("c")
        b_iotas = [
            lax.iota(jnp.int32, _L) + j * _L for j in range(_BT // _L)
        ]

        def do_block(n1, nf):
            pltpu.sync_copy(xq_hbm.at[n1, w], idx_v)
            pltpu.sync_copy(vq_hbm.at[n1, w], val_v)
            if idx_scale != 1:
                for n0 in range(nf):
                    for j in range(_BT // _L):
                        idx_v[n0, pl.ds(j * _L, _L)] = (
                            idx_v[n0, pl.ds(j * _L, _L)] * idx_scale
                        )
            copies = [
                pltpu.async_copy(
                    table_hbm.at[idx_v.at[n0]], rows_v.at[n0], gsem
                )
                for n0 in range(nf)
            ]
            for cp in copies:
                cp.wait()

            def n0_body(n0, carry):
                n0v = jnp.full((_L,), n0, jnp.int32)
                for j in range(_BT // _L):
                    vj = val_v[n0, pl.ds(j * _L, _L)]
                    bj = b_iotas[j]
                    for c in range(D):
                        vals = plsc.load_gather(
                            rows_v,
                            [n0v, bj, jnp.full((_L,), c, jnp.int32)],
                        )
                        out_v[n0, c // _FT, c % _FT, pl.ds(j * _L, _L)] = (
                            vals * vj
                        )
                return carry

            lax.fori_loop(0, nf, n0_body, 0)

            ocopies = [
                pltpu.async_copy(
                    out_v.at[n0, c1],
                    out_hbm.at[n1 * _FT + n0, c1, w],
                    osem,
                )
                for n0 in range(nf)
                for c1 in range(c_tiles)
            ]
            for cp in ocopies:
                cp.wait()

        def blk_body(n1, carry):
            do_block(n1, _FT)
            return carry

        lax.fori_loop(0, n_blocks, blk_body, 0)
        if n_tail:
            do_block(n_blocks, n_tail)

    return k


@jax.jit
def kernel(x, x_val, table):
    B, NNZ = x.shape
    V, D = table.shape
    nt = (NNZ + _FT - 1) // _FT
    pad = nt * _FT - NNZ

    def to_phys(a):
        ap = jnp.pad(a.T, ((0, pad), (0, 0)))
        return ap.reshape(nt, _FT, B // _BT, _BT).transpose(0, 2, 1, 3)

    q = jnp.pad(table, ((0, 0), (0, 128 - D)))
    tbl = q.reshape(V * (128 // D), D)
    out5 = _make_kernel(B, NNZ, V, D, 128 // D)(
        tbl, to_phys(x.astype(jnp.int32)), to_phys(x_val)
    )
    return out5.transpose(2, 4, 0, 1, 3).reshape(B, NNZ, D)


# speedup vs baseline: 1.1295x; 1.1295x over previous
import functools

import jax
import jax.numpy as jnp
from jax import lax
from jax.experimental import pallas as pl
from jax.experimental.pallas import tpu as pltpu
from jax.experimental.pallas import tpu_sc as plsc

_INFO = plsc.get_sparse_core_info()
_NC, _NS, _L = _INFO.num_cores, _INFO.num_subcores, _INFO.num_lanes
_NW = _NC * _NS

_G = 16


def _make_kernel(B, NNZ, V, D):
    assert B % _NW == 0
    rows_per_w = B // _NW
    assert rows_per_w % _G == 0
    n_chunks = rows_per_w // _G
    n_full = NNZ // _L
    n_tail = NNZ - n_full * _L

    @functools.partial(
        pl.kernel,
        out_type=jax.ShapeDtypeStruct((B, NNZ, D), jnp.float32),
        mesh=plsc.VectorSubcoreMesh(core_axis_name="c", subcore_axis_name="s"),
        scratch_types=[
            pltpu.VMEM((_G, NNZ), jnp.int32),
            pltpu.VMEM((_G, NNZ), jnp.float32),
            pltpu.VMEM((_G, NNZ, D), jnp.float32),
            pltpu.SemaphoreType.DMA,
        ],
        compiler_params=pltpu.CompilerParams(
            use_tc_tiling_on_sc=False, needs_layout_passes=False
        ),
    )
    def k(table_hbm, x_hbm, xval_hbm, out_hbm, idx_v, xval_v, rows_v, sem):
        wid = lax.axis_index("s") * _NC + lax.axis_index("c")
        base = wid * rows_per_w

        def scale_rows(g):
            def do_row(n, s):
                for c in range(D // _L):
                    rows_v[g, n, pl.ds(c * _L, _L)] = (
                        rows_v[g, n, pl.ds(c * _L, _L)] * s
                    )

            for h in range(n_full):
                sv = xval_v[g, pl.ds(h * _L, _L)]
                for kk in range(_L):
                    do_row(h * _L + kk, jnp.full((_L,), sv[kk], jnp.float32))
            if n_tail:
                nv = jnp.minimum(
                    lax.iota(jnp.int32, _L) + (NNZ - n_tail), NNZ - 1
                )
                gv = jnp.full((_L,), g, jnp.int32)
                sv = plsc.load_gather(xval_v, [gv, nv])
                for kk in range(n_tail):
                    do_row(
                        NNZ - n_tail + kk,
                        jnp.full((_L,), sv[kk], jnp.float32),
                    )

        def chunk_body(ci, carry):
            b0 = base + ci * _G
            pltpu.sync_copy(x_hbm.at[pl.ds(b0, _G)], idx_v)
            pltpu.sync_copy(xval_hbm.at[pl.ds(b0, _G)], xval_v)
            copies = [
                pltpu.async_copy(
                    table_hbm.at[idx_v.at[g]], rows_v.at[g], sem
                )
                for g in range(_G)
            ]
            for cp in copies:
                cp.wait()

            def g_body(g, c2):
                scale_rows(g)
                return c2

            lax.fori_loop(0, _G, g_body, 0)
            pltpu.sync_copy(rows_v, out_hbm.at[pl.ds(b0, _G)])
            return carry

        lax.fori_loop(0, n_chunks, chunk_body, 0)

    return k


@jax.jit
def kernel(x, x_val, table):
    B, NNZ = x.shape
    V, D = table.shape
    return _make_kernel(B, NNZ, V, D)(table, x.astype(jnp.int32), x_val)
